# baseline (speedup 1.0000x reference)
import jax
import jax.numpy as jnp
from jax.experimental import pallas as pl
from jax.experimental.pallas import tpu as pltpu

B = 1024
IN = 784
H = 400
D = 256
K = 512

NC = 1
NJ = 4
BR = B // (NC * NJ)


def _mmb(a, b_t):
    return jax.lax.dot_general(a.astype(jnp.bfloat16),
                               b_t.astype(jnp.bfloat16),
                               (((1,), (1,)), ((), ())),
                               preferred_element_type=jnp.float32)


def _mm_hi(a, b_t):
    return jax.lax.dot_general(a, b_t, (((1,), (1,)), ((), ())),
                               preferred_element_type=jnp.float32,
                               precision=jax.lax.Precision.HIGHEST)


def _vqvae_kernel(x_ref, fc1_w_ref, fc1_b_ref, fc2_w_ref, fc2_b_ref,
                  fc3_w_ref, fc3_b_ref, fc4_w_ref, fc4_b_ref, emb_ref,
                  xr_ref, rloss_ref, eloss_ref):
    f32 = jnp.float32
    j = pl.program_id(1)
    x = x_ref[...]
    h1 = jnp.maximum(_mmb(x, fc1_w_ref[...]) + fc1_b_ref[...], 0.0)
    z_e = _mmb(h1, fc2_w_ref[...]) + fc2_b_ref[...]
    emb = emb_ref[...]
    g = _mm_hi(z_e, emb)
    emb_sq = _mm_hi(jnp.ones((1, D), f32), emb * emb)
    score = emb_sq - 2.0 * g
    m = jnp.min(score, axis=1, keepdims=True)
    lane = jax.lax.broadcasted_iota(jnp.int32, (BR, K), 1)
    idx = jnp.min(jnp.where(score == m, lane, K), axis=1, keepdims=True)
    onehot = (lane == idx).astype(f32)
    z_q = jax.lax.dot_general(onehot, emb, (((1,), (0,)), ((), ())),
                              preferred_element_type=f32,
                              precision=jax.lax.Precision.HIGHEST)
    h3 = jnp.maximum(_mmb(z_q, fc3_w_ref[...]) + fc3_b_ref[...], 0.0)
    logits = _mmb(h3, fc4_w_ref[...]) + fc4_b_ref[...]
    x_reconst = jax.nn.sigmoid(logits)
    xr_ref[...] = x_reconst
    logp = jnp.maximum(jnp.log(x_reconst), -100.0)
    log1mp = jnp.maximum(jnp.log(1.0 - x_reconst), -100.0)
    rpart = -jnp.sum(x * logp + (1.0 - x) * log1mp) / (B * IN)
    dz = z_e - z_q
    epart = jnp.sum(dz * dz) / B

    @pl.when(j == 0)
    def _init():
        rloss_ref[...] = rpart[None, None, None]
        eloss_ref[...] = epart[None, None, None]

    @pl.when(j != 0)
    def _acc():
        rloss_ref[...] += rpart[None, None, None]
        eloss_ref[...] += epart[None, None, None]


def kernel(x, fc1_w, fc1_b, fc2_w, fc2_b, fc3_w, fc3_b, fc4_w, fc4_b, emb):
    row = lambda c, j: (c * NJ + j, 0)
    rep = lambda c, j: (0, 0)
    out = pl.pallas_call(
        _vqvae_kernel,
        grid=(NC, NJ),
        in_specs=[
            pl.BlockSpec((BR, IN), row),
            pl.BlockSpec((H, IN), rep),
            pl.BlockSpec((1, H), rep),
            pl.BlockSpec((D, H), rep),
            pl.BlockSpec((1, D), rep),
            pl.BlockSpec((H, D), rep),
            pl.BlockSpec((1, H), rep),
            pl.BlockSpec((IN, H), rep),
            pl.BlockSpec((1, IN), rep),
            pl.BlockSpec((K, D), rep),
        ],
        out_specs=(
            pl.BlockSpec((BR, IN), row),
            pl.BlockSpec((1, 1, 1), lambda c, j: (c, 0, 0)),
            pl.BlockSpec((1, 1, 1), lambda c, j: (c, 0, 0)),
        ),
        out_shape=(
            jax.ShapeDtypeStruct((B, IN), jnp.float32),
            jax.ShapeDtypeStruct((NC, 1, 1), jnp.float32),
            jax.ShapeDtypeStruct((NC, 1, 1), jnp.float32),
        ),
        compiler_params=pltpu.CompilerParams(
            dimension_semantics=("parallel", "arbitrary")),
    )(x, fc1_w, fc1_b.reshape(1, H), fc2_w, fc2_b.reshape(1, D),
      fc3_w, fc3_b.reshape(1, H), fc4_w, fc4_b.reshape(1, IN), emb)
    x_reconst, rloss, eloss = out
    return (x_reconst, jnp.sum(rloss), jnp.sum(eloss), jnp.sum(eloss))


# speedup vs baseline: 1.2080x; 1.2080x over previous
import jax
import jax.numpy as jnp
from jax.experimental import pallas as pl

B = 1024
IN = 784
H = 400
D = 256
K = 512


def _mmb(a, b_t):
    return jax.lax.dot_general(a.astype(jnp.bfloat16),
                               b_t.astype(jnp.bfloat16),
                               (((1,), (1,)), ((), ())),
                               preferred_element_type=jnp.float32)


def _mm_hi(a, b_t):
    return jax.lax.dot_general(a, b_t, (((1,), (1,)), ((), ())),
                               preferred_element_type=jnp.float32,
                               precision=jax.lax.Precision.HIGHEST)


def _vqvae_kernel(x_ref, fc1_w_ref, fc1_b_ref, fc2_w_ref, fc2_b_ref,
                  fc3_w_ref, fc3_b_ref, fc4_w_ref, fc4_b_ref, emb_ref,
                  xr_ref, rloss_ref, eloss_ref):
    f32 = jnp.float32
    x = x_ref[...]
    h1 = jnp.maximum(_mmb(x, fc1_w_ref[...]) + fc1_b_ref[...], 0.0)
    z_e = _mmb(h1, fc2_w_ref[...]) + fc2_b_ref[...]
    emb = emb_ref[...]
    g = _mm_hi(z_e, emb)
    emb_sq = _mm_hi(jnp.ones((1, D), f32), emb * emb)
    score = emb_sq - 2.0 * g
    m = jnp.min(score, axis=1, keepdims=True)
    lane = jax.lax.broadcasted_iota(jnp.int32, (B, K), 1)
    idx = jnp.min(jnp.where(score == m, lane, K), axis=1, keepdims=True)
    onehot = (lane == idx).astype(jnp.bfloat16)
    z_q = jax.lax.dot_general(onehot, emb.astype(jnp.bfloat16),
                              (((1,), (0,)), ((), ())),
                              preferred_element_type=f32)
    h3 = jnp.maximum(_mmb(z_q, fc3_w_ref[...]) + fc3_b_ref[...], 0.0)
    logits = _mmb(h3, fc4_w_ref[...]) + fc4_b_ref[...]
    en = jnp.exp(-logits)
    x_reconst = 1.0 / (1.0 + en)
    xr_ref[...] = x_reconst
    c = jnp.log(1.0 + en)
    logp = jnp.maximum(-c, -100.0)
    log1mp = jnp.maximum(-logits - c, -100.0)
    rloss = -jnp.sum(x * logp + (1.0 - x) * log1mp) / (B * IN)
    rloss_ref[...] = rloss[None, None]
    dz = z_e - z_q
    eloss = jnp.sum(dz * dz) / B
    eloss_ref[...] = eloss[None, None]


def kernel(x, fc1_w, fc1_b, fc2_w, fc2_b, fc3_w, fc3_b, fc4_w, fc4_b, emb):
    out = pl.pallas_call(
        _vqvae_kernel,
        out_shape=(
            jax.ShapeDtypeStruct((B, IN), jnp.float32),
            jax.ShapeDtypeStruct((1, 1), jnp.float32),
            jax.ShapeDtypeStruct((1, 1), jnp.float32),
        ),
    )(x, fc1_w, fc1_b.reshape(1, H), fc2_w, fc2_b.reshape(1, D),
      fc3_w, fc3_b.reshape(1, H), fc4_w, fc4_b.reshape(1, IN), emb)
    x_reconst, rloss, eloss = out
    rl = rloss[0, 0]
    el = eloss[0, 0]
    return (x_reconst, rl, el, el)
